# baseline (speedup 1.0000x reference)
import jax
import jax.numpy as jnp
from jax.experimental import pallas as pl

DIM = 32
LEVELS = 4
CB = 8192
BETA = 0.25
TOKEN_TILE = 128


def _rvq_body(x_ref, *refs):
    wT_refs = refs[0:4]
    w_refs = refs[4:8]
    wsq_refs = refs[8:12]
    quant_ref = refs[12]
    code_refs = refs[13:17]
    lsum_ref = refs[17]

    T = x_ref.shape[0]
    res = x_ref[...]
    quant = jnp.zeros_like(res)
    iota = jax.lax.broadcasted_iota(jnp.int32, (T, CB), 1)
    lane_iota = jax.lax.broadcasted_iota(jnp.int32, (1, 128), 1)
    lvec = jnp.zeros((1, 128), dtype=jnp.float32)

    for l in range(LEVELS):
        wT = wT_refs[l][...]
        w = w_refs[l][...]
        wsq = wsq_refs[l][...]
        xsq = jnp.sum(res * res, axis=1, keepdims=True)
        r_hi = res.astype(jnp.bfloat16)
        r_lo = (res - r_hi.astype(jnp.float32)).astype(jnp.bfloat16)
        lhs = jnp.concatenate([r_hi, r_hi, r_lo], axis=1)
        g = jax.lax.dot_general(lhs, wT, (((1,), (0,)), ((), ())),
                                preferred_element_type=jnp.float32)
        dist = xsq - 2.0 * g
        dist = dist + wsq
        minval = jnp.min(dist, axis=1, keepdims=True)
        idx = jnp.min(jnp.where(dist == minval, iota, CB), axis=1,
                      keepdims=True)
        onehot = (iota == idx).astype(jnp.float32)
        q = jax.lax.dot_general(onehot, w, (((1,), (0,)), ((), ())),
                                preferred_element_type=jnp.float32,
                                precision=jax.lax.Precision.HIGHEST)
        code_refs[l][...] = idx
        quant = quant + q
        res = res - q
        s = jnp.sum(res * res)
        lvec = lvec + jnp.where(lane_iota == l, s, 0.0)

    quant_ref[...] = quant

    @pl.when(pl.program_id(0) == 0)
    def _init():
        lsum_ref[...] = jnp.zeros_like(lsum_ref)

    lsum_ref[...] += lvec


@jax.jit
def _rvq(x, w0, w1, w2, w3):
    B, S, D = x.shape
    N = B * S
    T = TOKEN_TILE
    NT = N // T
    flat = x.reshape(N, D)
    ws = [w0, w1, w2, w3]

    def _split96(w):
        wT = w.T
        hi = wT.astype(jnp.bfloat16)
        lo = (wT - hi.astype(jnp.float32)).astype(jnp.bfloat16)
        return jnp.concatenate([hi, lo, hi], axis=0)

    wTs = [_split96(w) for w in ws]
    wsqs = [jnp.sum(w ** 2, axis=1).reshape(1, CB) for w in ws]

    full = lambda i: (0, 0)
    in_specs = (
        [pl.BlockSpec((T, D), lambda i: (i, 0))]
        + [pl.BlockSpec((3 * D, CB), full) for _ in range(LEVELS)]
        + [pl.BlockSpec((CB, D), full) for _ in range(LEVELS)]
        + [pl.BlockSpec((1, CB), full) for _ in range(LEVELS)]
    )
    out_specs = (
        [pl.BlockSpec((T, D), lambda i: (i, 0))]
        + [pl.BlockSpec((T, 1), lambda i: (i, 0)) for _ in range(LEVELS)]
        + [pl.BlockSpec((1, 128), full)]
    )
    out_shape = (
        [jax.ShapeDtypeStruct((N, D), jnp.float32)]
        + [jax.ShapeDtypeStruct((N, 1), jnp.int32) for _ in range(LEVELS)]
        + [jax.ShapeDtypeStruct((1, 128), jnp.float32)]
    )

    outs = pl.pallas_call(
        _rvq_body,
        grid=(NT,),
        in_specs=in_specs,
        out_specs=out_specs,
        out_shape=out_shape,
    )(flat, *wTs, *ws, *wsqs)

    quant = outs[0].reshape(B, S, D)
    codes = tuple(outs[1 + l].reshape(B, S) for l in range(LEVELS))
    lsums = outs[1 + LEVELS][0, :LEVELS]
    loss = jnp.sum(lsums * ((1.0 + BETA) / (N * D))).astype(jnp.float32)
    return quant, codes, loss


def kernel(x, w0, w1, w2, w3):
    return _rvq(x, w0, w1, w2, w3)
